# Initial kernel scaffold; baseline (speedup 1.0000x reference)
import jax
import jax.numpy as jnp
from jax.experimental import pallas as pl


def kernel(x, Wg, W1, W3, W2):
    raise NotImplementedError("write your pallas kernel here")


# speedup vs baseline: 7.0097x; 7.0097x over previous
import functools

import jax
import jax.numpy as jnp
from jax import lax
from jax.experimental import pallas as pl
from jax.experimental.pallas import tpu as pltpu
from jax.experimental.pallas import tpu_sc as plsc

T = 4096
D = 768
E = 64
DFF = 768
BM = 128
NBMAX = 128
P = NBMAX * BM
TB = 512
NTB = T // TB
NW = 32
TPW = T // NW
CHT = 32
NEG = -1e30


def _router_body(x_ref, wg_ref, logits_ref, e1_ref, e2_ref, r1_ref, r2_ref,
                 w1_ref, w2_ref, pc_ref):
    x = x_ref[...]
    wg = wg_ref[...]
    logits = lax.dot_general(x, wg, (((1,), (1,)), ((), ())),
                             preferred_element_type=jnp.float32)
    logits_ref[...] = logits

    iota = lax.broadcasted_iota(jnp.int32, (TB, E), 1)
    m1 = jnp.max(logits, axis=1, keepdims=True)
    i1 = jnp.min(jnp.where(logits == m1, iota, E), axis=1)
    l2 = jnp.where(iota == i1[:, None], NEG, logits)
    m2 = jnp.max(l2, axis=1, keepdims=True)
    i2 = jnp.min(jnp.where(l2 == m2, iota, E), axis=1)

    w1 = 1.0 / (1.0 + jnp.exp(m2 - m1))
    e1_ref[0, 0, :] = i1
    e2_ref[0, 0, :] = i2
    w1_ref[0, 0, :] = w1[:, 0]
    w2_ref[0, 0, :] = 1.0 - w1[:, 0]

    oh1 = (iota == i1[:, None]).astype(jnp.float32)
    oh2 = (iota == i2[:, None]).astype(jnp.float32)
    s = oh1 + oh2
    ri = lax.broadcasted_iota(jnp.int32, (TB, TB), 0)
    ci = lax.broadcasted_iota(jnp.int32, (TB, TB), 1)
    tri = (ri > ci).astype(jnp.float32)
    c = lax.dot_general(tri, s, (((1,), (0,)), ((), ())),
                        preferred_element_type=jnp.float32)
    r1 = jnp.sum(c * oh1, axis=1)
    r2 = jnp.sum((c + oh1) * oh2, axis=1)
    r1_ref[0, 0, :] = r1.astype(jnp.int32)
    r2_ref[0, 0, :] = r2.astype(jnp.int32)
    pc_ref[0, 0, :] = jnp.sum(s, axis=0).astype(jnp.int32)


def _router(xf, Wg):
    out_shapes = (
        jax.ShapeDtypeStruct((T, E), jnp.float32),
        jax.ShapeDtypeStruct((NTB, 1, TB), jnp.int32),
        jax.ShapeDtypeStruct((NTB, 1, TB), jnp.int32),
        jax.ShapeDtypeStruct((NTB, 1, TB), jnp.int32),
        jax.ShapeDtypeStruct((NTB, 1, TB), jnp.int32),
        jax.ShapeDtypeStruct((NTB, 1, TB), jnp.float32),
        jax.ShapeDtypeStruct((NTB, 1, TB), jnp.float32),
        jax.ShapeDtypeStruct((NTB, 1, E), jnp.int32),
    )
    tok3 = pl.BlockSpec((1, 1, TB), lambda g: (g, 0, 0))
    return pl.pallas_call(
        _router_body,
        grid=(NTB,),
        in_specs=[
            pl.BlockSpec((TB, D), lambda g: (g, 0)),
            pl.BlockSpec((E, D), lambda g: (0, 0)),
        ],
        out_specs=(
            pl.BlockSpec((TB, E), lambda g: (g, 0)),
            tok3, tok3, tok3, tok3, tok3, tok3,
            pl.BlockSpec((1, 1, E), lambda g: (g, 0, 0)),
        ),
        out_shape=out_shapes,
    )(xf, Wg)


def _dispatch_body(xf_hbm, e1_hbm, e2_hbm, r1_hbm, r2_hbm, pc_hbm,
                   xpad_hbm, pos1_hbm, pos2_hbm, bexp_hbm, nbt_hbm,
                   xrows, ev1, ev2, rv1, rv2, pv1, pv2,
                   pcv, pov, carryv, sbv, cntv, bexpv, nbtv, sem1, sem2):
    w = lax.axis_index("s") * 2 + lax.axis_index("c")
    base = w * TPW
    b0 = w // (NW // NTB)

    pltpu.sync_copy(e1_hbm.at[pl.ds(base, TPW)], ev1)
    pltpu.sync_copy(e2_hbm.at[pl.ds(base, TPW)], ev2)
    pltpu.sync_copy(r1_hbm.at[pl.ds(base, TPW)], rv1)
    pltpu.sync_copy(r2_hbm.at[pl.ds(base, TPW)], rv2)
    pltpu.sync_copy(pc_hbm, pcv)
    pltpu.sync_copy(xf_hbm.at[pl.ds(base, TPW)], xrows)

    for ch in range(E // 16):
        tot = jnp.zeros((16,), jnp.int32)
        car = jnp.zeros((16,), jnp.int32)
        for b in range(NTB):
            v = pcv[pl.ds(b * E + ch * 16, 16)]
            flag = jnp.where(b < b0, 1, 0).astype(jnp.int32)
            tot = tot + v
            car = car + v * flag
        sbv[pl.ds(ch * 16, 16)] = tot
        carryv[pl.ds(ch * 16, 16)] = car

    prev = jnp.zeros((), jnp.int32)
    for ch in range(E // 16):
        tot = sbv[pl.ds(ch * 16, 16)]
        nb = (tot + (BM - 1)) // BM
        inc = plsc.cumsum(nb)
        sb = inc - nb + prev
        sbv[pl.ds(ch * 16, 16)] = sb
        pov[pl.ds(ch * 16, 16)] = sb * BM
        prev = prev + jnp.sum(nb)
    nbt = prev

    for ch in range(TPW // 16):
        sl = pl.ds(ch * 16, 16)
        ids1 = ev1[sl]
        ids2 = ev2[sl]
        po1 = plsc.load_gather(pov, [ids1])
        po2 = plsc.load_gather(pov, [ids2])
        ca1 = plsc.load_gather(carryv, [ids1])
        ca2 = plsc.load_gather(carryv, [ids2])
        pv1[sl] = po1 + ca1 + rv1[sl]
        pv2[sl] = po2 + ca2 + rv2[sl]

    pltpu.sync_copy(pv1, pos1_hbm.at[pl.ds(base, TPW)])
    pltpu.sync_copy(pv2, pos2_hbm.at[pl.ds(base, TPW)])

    cp1 = pltpu.make_async_copy(xrows, xpad_hbm.at[pv1], sem1)
    cp2 = pltpu.make_async_copy(xrows, xpad_hbm.at[pv2], sem2)
    cp1.start()
    cp2.start()
    cp1.wait()
    cp2.wait()

    @pl.when(w == 0)
    def _():
        for ch in range(NBMAX // 16):
            cntv[pl.ds(ch * 16, 16)] = jnp.zeros((16,), jnp.int32)
        for ch in range(E // 16):
            plsc.addupdate_scatter(cntv, [sbv[pl.ds(ch * 16, 16)]],
                                   jnp.ones((16,), jnp.int32))
        prev2 = jnp.zeros((), jnp.int32)
        for ch in range(NBMAX // 16):
            v = cntv[pl.ds(ch * 16, 16)]
            inc = plsc.cumsum(v)
            bexpv[pl.ds(ch * 16, 16)] = inc + prev2 - 1
            prev2 = prev2 + jnp.sum(v)
        nbtv[pl.ds(0, 16)] = jnp.zeros((16,), jnp.int32) + nbt
        pltpu.sync_copy(bexpv, bexp_hbm)
        pltpu.sync_copy(nbtv, nbt_hbm)


def _dispatch(xf, e1, e2, r1, r2, pc):
    mesh = plsc.VectorSubcoreMesh(core_axis_name="c", subcore_axis_name="s", num_cores=2, num_subcores=16)
    fn = pl.kernel(
        _dispatch_body,
        out_type=(
            jax.ShapeDtypeStruct((P, D), jnp.float32),
            jax.ShapeDtypeStruct((T,), jnp.int32),
            jax.ShapeDtypeStruct((T,), jnp.int32),
            jax.ShapeDtypeStruct((NBMAX,), jnp.int32),
            jax.ShapeDtypeStruct((16,), jnp.int32),
        ),
        mesh=mesh,
        compiler_params=pltpu.CompilerParams(needs_layout_passes=False),
        scratch_types=[
            pltpu.VMEM((TPW, D), jnp.float32),
            pltpu.VMEM((TPW,), jnp.int32),
            pltpu.VMEM((TPW,), jnp.int32),
            pltpu.VMEM((TPW,), jnp.int32),
            pltpu.VMEM((TPW,), jnp.int32),
            pltpu.VMEM((TPW,), jnp.int32),
            pltpu.VMEM((TPW,), jnp.int32),
            pltpu.VMEM((NTB * E,), jnp.int32),
            pltpu.VMEM((E,), jnp.int32),
            pltpu.VMEM((E,), jnp.int32),
            pltpu.VMEM((E,), jnp.int32),
            pltpu.VMEM((NBMAX,), jnp.int32),
            pltpu.VMEM((NBMAX,), jnp.int32),
            pltpu.VMEM((16,), jnp.int32),
            pltpu.SemaphoreType.DMA,
            pltpu.SemaphoreType.DMA,
        ],
    )
    return fn(xf, e1, e2, r1, r2, pc)


def _mlp_body(bexp_ref, nbt_ref, x_ref, w1_ref, w3_ref, w2_ref, out_ref):
    @pl.when(pl.program_id(0) < nbt_ref[0])
    def _():
        x = x_ref[...]
        h1 = lax.dot_general(x, w1_ref[0], (((1,), (1,)), ((), ())),
                             preferred_element_type=jnp.float32)
        h3 = lax.dot_general(x, w3_ref[0], (((1,), (1,)), ((), ())),
                             preferred_element_type=jnp.float32)
        h = h1 * jax.nn.sigmoid(h1) * h3
        out_ref[...] = lax.dot_general(h, w2_ref[0], (((1,), (1,)), ((), ())),
                                       preferred_element_type=jnp.float32)


def _grouped_mlp(bexp, nbt, Xpad, W1, W3, W2):
    grid_spec = pltpu.PrefetchScalarGridSpec(
        num_scalar_prefetch=2,
        grid=(NBMAX,),
        in_specs=[
            pl.BlockSpec((BM, D), lambda g, be, nb: (g, 0)),
            pl.BlockSpec((1, DFF, D), lambda g, be, nb: (be[g], 0, 0)),
            pl.BlockSpec((1, DFF, D), lambda g, be, nb: (be[g], 0, 0)),
            pl.BlockSpec((1, D, DFF), lambda g, be, nb: (be[g], 0, 0)),
        ],
        out_specs=pl.BlockSpec((BM, D), lambda g, be, nb: (g, 0)),
    )
    return pl.pallas_call(
        _mlp_body,
        grid_spec=grid_spec,
        out_shape=jax.ShapeDtypeStruct((P, D), jnp.float32),
    )(bexp, nbt, Xpad, W1, W3, W2)


def _combine_body(out_hbm, pos1_hbm, pos2_hbm, w1_hbm, w2_hbm, fin_hbm,
                  p1c, p2c, wac, wbc, rows_a, rows_b, sem):
    w = lax.axis_index("s") * 2 + lax.axis_index("c")
    for chunk in range(TPW // CHT):
        tbase = w * TPW + chunk * CHT
        pltpu.sync_copy(pos1_hbm.at[pl.ds(tbase, CHT)], p1c)
        pltpu.sync_copy(pos2_hbm.at[pl.ds(tbase, CHT)], p2c)
        pltpu.sync_copy(w1_hbm.at[pl.ds(tbase, CHT)], wac)
        pltpu.sync_copy(w2_hbm.at[pl.ds(tbase, CHT)], wbc)
        cpa = pltpu.make_async_copy(out_hbm.at[p1c], rows_a, sem)
        cpa.start()
        cpa.wait()
        cpb = pltpu.make_async_copy(out_hbm.at[p2c], rows_b, sem)
        cpb.start()
        cpb.wait()

        def row_fma(r, _):
            ridx = jnp.zeros((16,), jnp.int32) + r
            wa = plsc.load_gather(wac, [ridx])
            wb = plsc.load_gather(wbc, [ridx])
            for v in range(D // 16):
                sl = pl.ds(v * 16, 16)
                rows_a[r, sl] = rows_a[r, sl] * wa + rows_b[r, sl] * wb
            return 0

        lax.fori_loop(0, CHT, row_fma, 0)
        pltpu.sync_copy(rows_a, fin_hbm.at[pl.ds(tbase, CHT)])


def _combine(Outpad, pos1, pos2, w1, w2):
    mesh = plsc.VectorSubcoreMesh(core_axis_name="c", subcore_axis_name="s", num_cores=2, num_subcores=16)
    fn = pl.kernel(
        _combine_body,
        out_type=jax.ShapeDtypeStruct((T, D), jnp.float32),
        mesh=mesh,
        compiler_params=pltpu.CompilerParams(needs_layout_passes=False),
        scratch_types=[
            pltpu.VMEM((CHT,), jnp.int32),
            pltpu.VMEM((CHT,), jnp.int32),
            pltpu.VMEM((CHT,), jnp.float32),
            pltpu.VMEM((CHT,), jnp.float32),
            pltpu.VMEM((CHT, D), jnp.float32),
            pltpu.VMEM((CHT, D), jnp.float32),
            pltpu.SemaphoreType.DMA,
        ],
    )
    return fn(Outpad, pos1, pos2, w1, w2)


@jax.jit
def kernel(x, Wg, W1, W3, W2):
    Bq, Sq, Dm = x.shape
    xf = x.reshape(-1, Dm)
    (logits, e1o, e2o, r1o, r2o, w1o, w2o, pco) = _router(xf, Wg)
    e1 = e1o.reshape(T)
    e2 = e2o.reshape(T)
    r1 = r1o.reshape(T)
    r2 = r2o.reshape(T)
    w1 = w1o.reshape(T)
    w2 = w2o.reshape(T)
    pc = pco.reshape(NTB * E)
    Xpad, pos1, pos2, bexp, nbt = _dispatch(xf, e1, e2, r1, r2, pc)
    Outpad = _grouped_mlp(bexp, nbt, Xpad, W1, W3, W2)
    final = _combine(Outpad, pos1, pos2, w1, w2)
    return final.reshape(Bq, Sq, Dm), logits


# speedup vs baseline: 7.2261x; 1.0309x over previous
import jax
import jax.numpy as jnp
from jax import lax
from jax.experimental import pallas as pl
from jax.experimental.pallas import tpu as pltpu
from jax.experimental.pallas import tpu_sc as plsc

T = 4096
D = 768
E = 64
DFF = 768
BM = 128
NBMAX = 128
P = NBMAX * BM
TB = 512
NTB = T // TB
NW = 32
TPW = T // NW
CHT = 32
NEG = -1e30


def _router_body(x_ref, wg_ref, logits_ref, e1_ref, e2_ref, r1_ref, r2_ref,
                 w1_ref, w2_ref, pc_ref):
    x = x_ref[...]
    wg = wg_ref[...]
    logits = lax.dot_general(x, wg, (((1,), (1,)), ((), ())),
                             preferred_element_type=jnp.float32)
    logits_ref[...] = logits

    iota = lax.broadcasted_iota(jnp.int32, (TB, E), 1)
    m1 = jnp.max(logits, axis=1, keepdims=True)
    i1 = jnp.min(jnp.where(logits == m1, iota, E), axis=1)
    l2 = jnp.where(iota == i1[:, None], NEG, logits)
    m2 = jnp.max(l2, axis=1, keepdims=True)
    i2 = jnp.min(jnp.where(l2 == m2, iota, E), axis=1)

    w1 = 1.0 / (1.0 + jnp.exp(m2 - m1))
    e1_ref[0, 0, :] = i1
    e2_ref[0, 0, :] = i2
    w1_ref[0, 0, :] = w1[:, 0]
    w2_ref[0, 0, :] = 1.0 - w1[:, 0]

    oh1 = (iota == i1[:, None]).astype(jnp.float32)
    oh2 = (iota == i2[:, None]).astype(jnp.float32)
    s = oh1 + oh2
    ri = lax.broadcasted_iota(jnp.int32, (TB, TB), 0)
    ci = lax.broadcasted_iota(jnp.int32, (TB, TB), 1)
    tri = (ri > ci).astype(jnp.float32)
    c = lax.dot_general(tri, s, (((1,), (0,)), ((), ())),
                        preferred_element_type=jnp.float32)
    r1 = jnp.sum(c * oh1, axis=1)
    r2 = jnp.sum((c + oh1) * oh2, axis=1)
    r1_ref[0, 0, :] = r1.astype(jnp.int32)
    r2_ref[0, 0, :] = r2.astype(jnp.int32)
    pc_ref[0, 0, :] = jnp.sum(s, axis=0).astype(jnp.int32)


def _router(xf, Wg):
    out_shapes = (
        jax.ShapeDtypeStruct((T, E), jnp.float32),
        jax.ShapeDtypeStruct((NTB, 1, TB), jnp.int32),
        jax.ShapeDtypeStruct((NTB, 1, TB), jnp.int32),
        jax.ShapeDtypeStruct((NTB, 1, TB), jnp.int32),
        jax.ShapeDtypeStruct((NTB, 1, TB), jnp.int32),
        jax.ShapeDtypeStruct((NTB, 1, TB), jnp.float32),
        jax.ShapeDtypeStruct((NTB, 1, TB), jnp.float32),
        jax.ShapeDtypeStruct((NTB, 1, E), jnp.int32),
    )
    tok3 = pl.BlockSpec((1, 1, TB), lambda g: (g, 0, 0))
    return pl.pallas_call(
        _router_body,
        grid=(NTB,),
        in_specs=[
            pl.BlockSpec((TB, D), lambda g: (g, 0)),
            pl.BlockSpec((E, D), lambda g: (0, 0)),
        ],
        out_specs=(
            pl.BlockSpec((TB, E), lambda g: (g, 0)),
            tok3, tok3, tok3, tok3, tok3, tok3,
            pl.BlockSpec((1, 1, E), lambda g: (g, 0, 0)),
        ),
        out_shape=out_shapes,
    )(xf, Wg)


def _dispatch_body(xf_hbm, e1_hbm, e2_hbm, r1_hbm, r2_hbm, pc_hbm,
                   xpad_hbm, pos1_hbm, pos2_hbm, bexp_hbm, nbt_hbm,
                   xrows, ev1, ev2, rv1, rv2, pv1, pv2,
                   pcv, pov, carryv, sbv, cntv, bexpv, nbtv, sem1, sem2):
    w = lax.axis_index("s") * 2 + lax.axis_index("c")
    base = w * TPW
    b0 = w // (NW // NTB)

    cpx = pltpu.make_async_copy(xf_hbm.at[pl.ds(base, TPW)], xrows, sem1)
    cpx.start()
    pltpu.sync_copy(e1_hbm.at[pl.ds(base, TPW)], ev1)
    pltpu.sync_copy(e2_hbm.at[pl.ds(base, TPW)], ev2)
    pltpu.sync_copy(r1_hbm.at[pl.ds(base, TPW)], rv1)
    pltpu.sync_copy(r2_hbm.at[pl.ds(base, TPW)], rv2)
    pltpu.sync_copy(pc_hbm, pcv)

    for ch in range(E // 16):
        tot = jnp.zeros((16,), jnp.int32)
        car = jnp.zeros((16,), jnp.int32)
        for b in range(NTB):
            v = pcv[pl.ds(b * E + ch * 16, 16)]
            flag = jnp.where(b < b0, 1, 0).astype(jnp.int32)
            tot = tot + v
            car = car + v * flag
        sbv[pl.ds(ch * 16, 16)] = tot
        carryv[pl.ds(ch * 16, 16)] = car

    prev = jnp.zeros((), jnp.int32)
    for ch in range(E // 16):
        tot = sbv[pl.ds(ch * 16, 16)]
        nb = (tot + (BM - 1)) // BM
        inc = plsc.cumsum(nb)
        sb = inc - nb + prev
        sbv[pl.ds(ch * 16, 16)] = sb
        pov[pl.ds(ch * 16, 16)] = sb * BM
        prev = prev + jnp.sum(nb)
    nbt = prev

    for ch in range(TPW // 16):
        sl = pl.ds(ch * 16, 16)
        ids1 = ev1[sl]
        ids2 = ev2[sl]
        po1 = plsc.load_gather(pov, [ids1])
        po2 = plsc.load_gather(pov, [ids2])
        ca1 = plsc.load_gather(carryv, [ids1])
        ca2 = plsc.load_gather(carryv, [ids2])
        pv1[sl] = po1 + ca1 + rv1[sl]
        pv2[sl] = po2 + ca2 + rv2[sl]

    pltpu.sync_copy(pv1, pos1_hbm.at[pl.ds(base, TPW)])
    pltpu.sync_copy(pv2, pos2_hbm.at[pl.ds(base, TPW)])

    cpx.wait()
    cp1 = pltpu.make_async_copy(xrows, xpad_hbm.at[pv1], sem1)
    cp2 = pltpu.make_async_copy(xrows, xpad_hbm.at[pv2], sem2)
    cp1.start()
    cp2.start()

    @pl.when(w == 0)
    def _():
        for ch in range(NBMAX // 16):
            cntv[pl.ds(ch * 16, 16)] = jnp.zeros((16,), jnp.int32)
        for ch in range(E // 16):
            plsc.addupdate_scatter(cntv, [sbv[pl.ds(ch * 16, 16)]],
                                   jnp.ones((16,), jnp.int32))
        prev2 = jnp.zeros((), jnp.int32)
        for ch in range(NBMAX // 16):
            v = cntv[pl.ds(ch * 16, 16)]
            inc = plsc.cumsum(v)
            bexpv[pl.ds(ch * 16, 16)] = inc + prev2 - 1
            prev2 = prev2 + jnp.sum(v)
        nbtv[pl.ds(0, 16)] = jnp.zeros((16,), jnp.int32) + nbt
        pltpu.sync_copy(bexpv, bexp_hbm)
        pltpu.sync_copy(nbtv, nbt_hbm)

    cp1.wait()
    cp2.wait()


def _dispatch(xf, e1, e2, r1, r2, pc):
    mesh = plsc.VectorSubcoreMesh(core_axis_name="c", subcore_axis_name="s", num_cores=2, num_subcores=16)
    fn = pl.kernel(
        _dispatch_body,
        out_type=(
            jax.ShapeDtypeStruct((P, D), jnp.float32),
            jax.ShapeDtypeStruct((T,), jnp.int32),
            jax.ShapeDtypeStruct((T,), jnp.int32),
            jax.ShapeDtypeStruct((NBMAX,), jnp.int32),
            jax.ShapeDtypeStruct((16,), jnp.int32),
        ),
        mesh=mesh,
        compiler_params=pltpu.CompilerParams(needs_layout_passes=False),
        scratch_types=[
            pltpu.VMEM((TPW, D), jnp.float32),
            pltpu.VMEM((TPW,), jnp.int32),
            pltpu.VMEM((TPW,), jnp.int32),
            pltpu.VMEM((TPW,), jnp.int32),
            pltpu.VMEM((TPW,), jnp.int32),
            pltpu.VMEM((TPW,), jnp.int32),
            pltpu.VMEM((TPW,), jnp.int32),
            pltpu.VMEM((NTB * E,), jnp.int32),
            pltpu.VMEM((E,), jnp.int32),
            pltpu.VMEM((E,), jnp.int32),
            pltpu.VMEM((E,), jnp.int32),
            pltpu.VMEM((NBMAX,), jnp.int32),
            pltpu.VMEM((NBMAX,), jnp.int32),
            pltpu.VMEM((16,), jnp.int32),
            pltpu.SemaphoreType.DMA,
            pltpu.SemaphoreType.DMA,
        ],
    )
    return fn(xf, e1, e2, r1, r2, pc)


def _mlp_body(bexp_ref, nbt_ref, x_ref, w1_ref, w3_ref, w2_ref, out_ref):
    @pl.when(pl.program_id(0) < nbt_ref[0])
    def _():
        x = x_ref[...]
        h1 = lax.dot_general(x, w1_ref[0], (((1,), (1,)), ((), ())),
                             preferred_element_type=jnp.float32)
        h3 = lax.dot_general(x, w3_ref[0], (((1,), (1,)), ((), ())),
                             preferred_element_type=jnp.float32)
        h = h1 * jax.nn.sigmoid(h1) * h3
        out_ref[...] = lax.dot_general(h, w2_ref[0], (((1,), (1,)), ((), ())),
                                       preferred_element_type=jnp.float32)


def _grouped_mlp(bexp, nbt, Xpad, W1, W3, W2):
    grid_spec = pltpu.PrefetchScalarGridSpec(
        num_scalar_prefetch=2,
        grid=(NBMAX,),
        in_specs=[
            pl.BlockSpec((BM, D), lambda g, be, nb: (jnp.minimum(g, nb[0] - 1), 0)),
            pl.BlockSpec((1, DFF, D), lambda g, be, nb: (be[g], 0, 0)),
            pl.BlockSpec((1, DFF, D), lambda g, be, nb: (be[g], 0, 0)),
            pl.BlockSpec((1, D, DFF), lambda g, be, nb: (be[g], 0, 0)),
        ],
        out_specs=pl.BlockSpec((BM, D), lambda g, be, nb: (g, 0)),
    )
    return pl.pallas_call(
        _mlp_body,
        grid_spec=grid_spec,
        out_shape=jax.ShapeDtypeStruct((P, D), jnp.float32),
    )(bexp, nbt, Xpad, W1, W3, W2)


def _combine_body(out_hbm, pos1_hbm, pos2_hbm, w1_hbm, w2_hbm, fin_hbm,
                  pv1, pv2, wv1, wv2, rows_a, rows_b, sems):
    w = lax.axis_index("s") * 2 + lax.axis_index("c")
    base = w * TPW
    pltpu.sync_copy(pos1_hbm.at[pl.ds(base, TPW)], pv1)
    pltpu.sync_copy(pos2_hbm.at[pl.ds(base, TPW)], pv2)
    pltpu.sync_copy(w1_hbm.at[pl.ds(base, TPW)], wv1)
    pltpu.sync_copy(w2_hbm.at[pl.ds(base, TPW)], wv2)

    nch = TPW // CHT

    def start(c):
        buf = c % 2
        sl = pl.ds(c * CHT, CHT)
        pltpu.make_async_copy(out_hbm.at[pv1.at[sl]], rows_a.at[buf],
                              sems.at[2 * buf]).start()
        pltpu.make_async_copy(out_hbm.at[pv2.at[sl]], rows_b.at[buf],
                              sems.at[2 * buf + 1]).start()

    start(0)
    for c in range(nch):
        buf = c % 2
        sl = pl.ds(c * CHT, CHT)
        if c + 1 < nch:
            start(c + 1)
        pltpu.make_async_copy(out_hbm.at[pv1.at[sl]], rows_a.at[buf],
                              sems.at[2 * buf]).wait()
        pltpu.make_async_copy(out_hbm.at[pv2.at[sl]], rows_b.at[buf],
                              sems.at[2 * buf + 1]).wait()

        def row_fma(r, _, buf=buf, c=c):
            ridx = jnp.zeros((16,), jnp.int32) + (c * CHT + r)
            wa = plsc.load_gather(wv1, [ridx])
            wb = plsc.load_gather(wv2, [ridx])
            for v in range(D // 16):
                sl = pl.ds(v * 16, 16)
                rows_a[buf, r, sl] = rows_a[buf, r, sl] * wa + rows_b[buf, r, sl] * wb
            return 0

        lax.fori_loop(0, CHT, row_fma, 0)
        pltpu.sync_copy(rows_a.at[buf], fin_hbm.at[pl.ds(base + c * CHT, CHT)])


def _combine(Outpad, pos1, pos2, w1, w2):
    mesh = plsc.VectorSubcoreMesh(core_axis_name="c", subcore_axis_name="s", num_cores=2, num_subcores=16)
    fn = pl.kernel(
        _combine_body,
        out_type=jax.ShapeDtypeStruct((T, D), jnp.float32),
        mesh=mesh,
        compiler_params=pltpu.CompilerParams(needs_layout_passes=False),
        scratch_types=[
            pltpu.VMEM((TPW,), jnp.int32),
            pltpu.VMEM((TPW,), jnp.int32),
            pltpu.VMEM((TPW,), jnp.float32),
            pltpu.VMEM((TPW,), jnp.float32),
            pltpu.VMEM((2, CHT, D), jnp.float32),
            pltpu.VMEM((2, CHT, D), jnp.float32),
            pltpu.SemaphoreType.DMA((4,)),
        ],
    )
    return fn(Outpad, pos1, pos2, w1, w2)


@jax.jit
def kernel(x, Wg, W1, W3, W2):
    Bq, Sq, Dm = x.shape
    xf = x.reshape(-1, Dm)
    (logits, e1o, e2o, r1o, r2o, w1o, w2o, pco) = _router(xf, Wg)
    e1 = e1o.reshape(T)
    e2 = e2o.reshape(T)
    r1 = r1o.reshape(T)
    r2 = r2o.reshape(T)
    w1 = w1o.reshape(T)
    w2 = w2o.reshape(T)
    pc = pco.reshape(NTB * E)
    Xpad, pos1, pos2, bexp, nbt = _dispatch(xf, e1, e2, r1, r2, pc)
    Outpad = _grouped_mlp(bexp, nbt, Xpad, W1, W3, W2)
    final = _combine(Outpad, pos1, pos2, w1, w2)
    return final.reshape(Bq, Sq, Dm), logits


# speedup vs baseline: 7.4522x; 1.0313x over previous
import jax
import jax.numpy as jnp
from jax import lax
from jax.experimental import pallas as pl
from jax.experimental.pallas import tpu as pltpu
from jax.experimental.pallas import tpu_sc as plsc

T = 4096
D = 768
E = 64
DFF = 768
BM = 128
NBMAX = 128
P = NBMAX * BM
TB = 512
NTB = T // TB
NW = 32
TPW = T // NW
CHT = 32
NEG = -1e30


def _router_body(x_ref, wg_ref, logits_ref, e1_ref, e2_ref, r1_ref, r2_ref,
                 w1_ref, w2_ref, pc_ref):
    x = x_ref[...]
    wg = wg_ref[...]
    logits = lax.dot_general(x, wg, (((1,), (1,)), ((), ())),
                             preferred_element_type=jnp.float32)
    logits_ref[...] = logits

    iota = lax.broadcasted_iota(jnp.int32, (TB, E), 1)
    m1 = jnp.max(logits, axis=1, keepdims=True)
    i1 = jnp.min(jnp.where(logits == m1, iota, E), axis=1)
    l2 = jnp.where(iota == i1[:, None], NEG, logits)
    m2 = jnp.max(l2, axis=1, keepdims=True)
    i2 = jnp.min(jnp.where(l2 == m2, iota, E), axis=1)

    w1 = 1.0 / (1.0 + jnp.exp(m2 - m1))
    e1_ref[0, 0, :] = i1
    e2_ref[0, 0, :] = i2
    w1_ref[0, 0, :] = w1[:, 0]
    w2_ref[0, 0, :] = 1.0 - w1[:, 0]

    oh1 = (iota == i1[:, None]).astype(jnp.float32)
    oh2 = (iota == i2[:, None]).astype(jnp.float32)
    s = oh1 + oh2
    ri = lax.broadcasted_iota(jnp.int32, (TB, TB), 0)
    ci = lax.broadcasted_iota(jnp.int32, (TB, TB), 1)
    tri = (ri > ci).astype(jnp.float32)
    c = lax.dot_general(tri, s, (((1,), (0,)), ((), ())),
                        preferred_element_type=jnp.float32)
    r1 = jnp.sum(c * oh1, axis=1)
    r2 = jnp.sum((c + oh1) * oh2, axis=1)
    r1_ref[0, 0, :] = r1.astype(jnp.int32)
    r2_ref[0, 0, :] = r2.astype(jnp.int32)
    pc_ref[0, 0, :] = jnp.sum(s, axis=0).astype(jnp.int32)


def _router(xf, Wg):
    out_shapes = (
        jax.ShapeDtypeStruct((T, E), jnp.float32),
        jax.ShapeDtypeStruct((NTB, 1, TB), jnp.int32),
        jax.ShapeDtypeStruct((NTB, 1, TB), jnp.int32),
        jax.ShapeDtypeStruct((NTB, 1, TB), jnp.int32),
        jax.ShapeDtypeStruct((NTB, 1, TB), jnp.int32),
        jax.ShapeDtypeStruct((NTB, 1, TB), jnp.float32),
        jax.ShapeDtypeStruct((NTB, 1, TB), jnp.float32),
        jax.ShapeDtypeStruct((NTB, 1, E), jnp.int32),
    )
    tok3 = pl.BlockSpec((1, 1, TB), lambda g: (g, 0, 0))
    return pl.pallas_call(
        _router_body,
        grid=(NTB,),
        in_specs=[
            pl.BlockSpec((TB, D), lambda g: (g, 0)),
            pl.BlockSpec((E, D), lambda g: (0, 0)),
        ],
        out_specs=(
            pl.BlockSpec((TB, E), lambda g: (g, 0)),
            tok3, tok3, tok3, tok3, tok3, tok3,
            pl.BlockSpec((1, 1, E), lambda g: (g, 0, 0)),
        ),
        out_shape=out_shapes,
    )(xf, Wg)


def _dispatch_body(xf_hbm, e1_hbm, e2_hbm, r1_hbm, r2_hbm, pc_hbm,
                   xpad_hbm, pos1_hbm, pos2_hbm, bexp_hbm, nbt_hbm,
                   xrows, ev1, ev2, rv1, rv2, pv1, pv2,
                   pcv, pov, carryv, sbv, cntv, bexpv, nbtv, sem1, sem2):
    w = lax.axis_index("s") * 2 + lax.axis_index("c")
    base = w * TPW
    b0 = w // (NW // NTB)

    cpx = pltpu.make_async_copy(xf_hbm.at[pl.ds(base, TPW)], xrows, sem1)
    cpx.start()
    pltpu.sync_copy(e1_hbm.at[pl.ds(base, TPW)], ev1)
    pltpu.sync_copy(e2_hbm.at[pl.ds(base, TPW)], ev2)
    pltpu.sync_copy(r1_hbm.at[pl.ds(base, TPW)], rv1)
    pltpu.sync_copy(r2_hbm.at[pl.ds(base, TPW)], rv2)
    pltpu.sync_copy(pc_hbm, pcv)

    for ch in range(E // 16):
        tot = jnp.zeros((16,), jnp.int32)
        car = jnp.zeros((16,), jnp.int32)
        for b in range(NTB):
            v = pcv[pl.ds(b * E + ch * 16, 16)]
            flag = jnp.where(b < b0, 1, 0).astype(jnp.int32)
            tot = tot + v
            car = car + v * flag
        sbv[pl.ds(ch * 16, 16)] = tot
        carryv[pl.ds(ch * 16, 16)] = car

    prev = jnp.zeros((), jnp.int32)
    for ch in range(E // 16):
        tot = sbv[pl.ds(ch * 16, 16)]
        nb = (tot + (BM - 1)) // BM
        inc = plsc.cumsum(nb)
        sb = inc - nb + prev
        sbv[pl.ds(ch * 16, 16)] = sb
        pov[pl.ds(ch * 16, 16)] = sb * BM
        prev = prev + jnp.sum(nb)
    nbt = prev

    for ch in range(TPW // 16):
        sl = pl.ds(ch * 16, 16)
        ids1 = ev1[sl]
        ids2 = ev2[sl]
        po1 = plsc.load_gather(pov, [ids1])
        po2 = plsc.load_gather(pov, [ids2])
        ca1 = plsc.load_gather(carryv, [ids1])
        ca2 = plsc.load_gather(carryv, [ids2])
        pv1[sl] = po1 + ca1 + rv1[sl]
        pv2[sl] = po2 + ca2 + rv2[sl]

    pltpu.sync_copy(pv1, pos1_hbm.at[pl.ds(base, TPW)])
    pltpu.sync_copy(pv2, pos2_hbm.at[pl.ds(base, TPW)])

    cpx.wait()
    cp1 = pltpu.make_async_copy(xrows, xpad_hbm.at[pv1], sem1)
    cp2 = pltpu.make_async_copy(xrows, xpad_hbm.at[pv2], sem2)
    cp1.start()
    cp2.start()

    @pl.when(w == 0)
    def _():
        for ch in range(NBMAX // 16):
            cntv[pl.ds(ch * 16, 16)] = jnp.zeros((16,), jnp.int32)
        for ch in range(E // 16):
            plsc.addupdate_scatter(cntv, [sbv[pl.ds(ch * 16, 16)]],
                                   jnp.ones((16,), jnp.int32))
        prev2 = jnp.zeros((), jnp.int32)
        for ch in range(NBMAX // 16):
            v = cntv[pl.ds(ch * 16, 16)]
            inc = plsc.cumsum(v)
            bexpv[pl.ds(ch * 16, 16)] = inc + prev2 - 1
            prev2 = prev2 + jnp.sum(v)
        nbtv[pl.ds(0, 16)] = jnp.zeros((16,), jnp.int32) + nbt
        pltpu.sync_copy(bexpv, bexp_hbm)
        pltpu.sync_copy(nbtv, nbt_hbm)

    cp1.wait()
    cp2.wait()


def _dispatch(xf, e1, e2, r1, r2, pc):
    mesh = plsc.VectorSubcoreMesh(core_axis_name="c", subcore_axis_name="s", num_cores=2, num_subcores=16)
    fn = pl.kernel(
        _dispatch_body,
        out_type=(
            jax.ShapeDtypeStruct((P, D), jnp.float32),
            jax.ShapeDtypeStruct((T,), jnp.int32),
            jax.ShapeDtypeStruct((T,), jnp.int32),
            jax.ShapeDtypeStruct((NBMAX,), jnp.int32),
            jax.ShapeDtypeStruct((16,), jnp.int32),
        ),
        mesh=mesh,
        compiler_params=pltpu.CompilerParams(needs_layout_passes=False),
        scratch_types=[
            pltpu.VMEM((TPW, D), jnp.float32),
            pltpu.VMEM((TPW,), jnp.int32),
            pltpu.VMEM((TPW,), jnp.int32),
            pltpu.VMEM((TPW,), jnp.int32),
            pltpu.VMEM((TPW,), jnp.int32),
            pltpu.VMEM((TPW,), jnp.int32),
            pltpu.VMEM((TPW,), jnp.int32),
            pltpu.VMEM((NTB * E,), jnp.int32),
            pltpu.VMEM((E,), jnp.int32),
            pltpu.VMEM((E,), jnp.int32),
            pltpu.VMEM((E,), jnp.int32),
            pltpu.VMEM((NBMAX,), jnp.int32),
            pltpu.VMEM((NBMAX,), jnp.int32),
            pltpu.VMEM((16,), jnp.int32),
            pltpu.SemaphoreType.DMA,
            pltpu.SemaphoreType.DMA,
        ],
    )
    return fn(xf, e1, e2, r1, r2, pc)


def _mlp_body(bexp_ref, nbt_ref, x_ref, w1_ref, w3_ref, w2_ref, out_ref):
    @pl.when(pl.program_id(0) < nbt_ref[0])
    def _():
        x = x_ref[...]
        h1 = lax.dot_general(x, w1_ref[0], (((1,), (1,)), ((), ())),
                             preferred_element_type=jnp.float32)
        h3 = lax.dot_general(x, w3_ref[0], (((1,), (1,)), ((), ())),
                             preferred_element_type=jnp.float32)
        h = h1 * jax.nn.sigmoid(h1) * h3
        out_ref[...] = lax.dot_general(h, w2_ref[0], (((1,), (1,)), ((), ())),
                                       preferred_element_type=jnp.float32)


def _grouped_mlp(bexp, nbt, Xpad, W1, W3, W2):
    grid_spec = pltpu.PrefetchScalarGridSpec(
        num_scalar_prefetch=2,
        grid=(NBMAX,),
        in_specs=[
            pl.BlockSpec((BM, D), lambda g, be, nb: (jnp.minimum(g, nb[0] - 1), 0)),
            pl.BlockSpec((1, DFF, D), lambda g, be, nb: (be[g], 0, 0)),
            pl.BlockSpec((1, DFF, D), lambda g, be, nb: (be[g], 0, 0)),
            pl.BlockSpec((1, D, DFF), lambda g, be, nb: (be[g], 0, 0)),
        ],
        out_specs=pl.BlockSpec((BM, D), lambda g, be, nb: (jnp.minimum(g, nb[0] - 1), 0)),
    )
    return pl.pallas_call(
        _mlp_body,
        grid_spec=grid_spec,
        out_shape=jax.ShapeDtypeStruct((P, D), jnp.float32),
    )(bexp, nbt, Xpad, W1, W3, W2)


def _combine_body(out_hbm, pos1_hbm, pos2_hbm, w1_hbm, w2_hbm, fin_hbm,
                  pv1, pv2, wv1, wv2, rows_a, rows_b, sems):
    w = lax.axis_index("s") * 2 + lax.axis_index("c")
    base = w * TPW
    pltpu.sync_copy(pos1_hbm.at[pl.ds(base, TPW)], pv1)
    pltpu.sync_copy(pos2_hbm.at[pl.ds(base, TPW)], pv2)
    pltpu.sync_copy(w1_hbm.at[pl.ds(base, TPW)], wv1)
    pltpu.sync_copy(w2_hbm.at[pl.ds(base, TPW)], wv2)

    nch = TPW // CHT

    def start(c):
        buf = c % 2
        sl = pl.ds(c * CHT, CHT)
        pltpu.make_async_copy(out_hbm.at[pv1.at[sl]], rows_a.at[buf],
                              sems.at[2 * buf]).start()
        pltpu.make_async_copy(out_hbm.at[pv2.at[sl]], rows_b.at[buf],
                              sems.at[2 * buf + 1]).start()

    start(0)
    for c in range(nch):
        buf = c % 2
        sl = pl.ds(c * CHT, CHT)
        if c + 1 < nch:
            start(c + 1)
        pltpu.make_async_copy(out_hbm.at[pv1.at[sl]], rows_a.at[buf],
                              sems.at[2 * buf]).wait()
        pltpu.make_async_copy(out_hbm.at[pv2.at[sl]], rows_b.at[buf],
                              sems.at[2 * buf + 1]).wait()

        def row_fma(r, _, buf=buf, c=c):
            ridx = jnp.zeros((16,), jnp.int32) + (c * CHT + r)
            wa = plsc.load_gather(wv1, [ridx])
            wb = plsc.load_gather(wv2, [ridx])
            for v in range(D // 16):
                sl = pl.ds(v * 16, 16)
                rows_a[buf, r, sl] = rows_a[buf, r, sl] * wa + rows_b[buf, r, sl] * wb
            return 0

        lax.fori_loop(0, CHT, row_fma, 0)
        pltpu.sync_copy(rows_a.at[buf], fin_hbm.at[pl.ds(base + c * CHT, CHT)])


def _combine(Outpad, pos1, pos2, w1, w2):
    mesh = plsc.VectorSubcoreMesh(core_axis_name="c", subcore_axis_name="s", num_cores=2, num_subcores=16)
    fn = pl.kernel(
        _combine_body,
        out_type=jax.ShapeDtypeStruct((T, D), jnp.float32),
        mesh=mesh,
        compiler_params=pltpu.CompilerParams(needs_layout_passes=False),
        scratch_types=[
            pltpu.VMEM((TPW,), jnp.int32),
            pltpu.VMEM((TPW,), jnp.int32),
            pltpu.VMEM((TPW,), jnp.float32),
            pltpu.VMEM((TPW,), jnp.float32),
            pltpu.VMEM((2, CHT, D), jnp.float32),
            pltpu.VMEM((2, CHT, D), jnp.float32),
            pltpu.SemaphoreType.DMA((4,)),
        ],
    )
    return fn(Outpad, pos1, pos2, w1, w2)


@jax.jit
def kernel(x, Wg, W1, W3, W2):
    Bq, Sq, Dm = x.shape
    xf = x.reshape(-1, Dm)
    (logits, e1o, e2o, r1o, r2o, w1o, w2o, pco) = _router(xf, Wg)
    e1 = e1o.reshape(T)
    e2 = e2o.reshape(T)
    r1 = r1o.reshape(T)
    r2 = r2o.reshape(T)
    w1 = w1o.reshape(T)
    w2 = w2o.reshape(T)
    pc = pco.reshape(NTB * E)
    Xpad, pos1, pos2, bexp, nbt = _dispatch(xf, e1, e2, r1, r2, pc)
    Outpad = _grouped_mlp(bexp, nbt, Xpad, W1, W3, W2)
    final = _combine(Outpad, pos1, pos2, w1, w2)
    return final.reshape(Bq, Sq, Dm), logits
